# baseline (speedup 1.0000x reference)
import functools

import jax
import jax.numpy as jnp
import numpy as np
from jax.experimental import pallas as pl
from jax.experimental.pallas import tpu as pltpu

_R = 6
_NEG = np.float32(-np.inf)


def _lexmax3(a, b):
    sa, ta, ka = a
    sb, tb, kb = b
    take = (sa > sb) | ((sa == sb) & (ta >= tb))
    return (jnp.where(take, sa, sb), jnp.where(take, ta, tb),
            jnp.where(take, ka, kb))


def _nms_body(pred_ref, w_ref, thr_ref, out_ref, st_a, st_b, st_c,
              *, B, H, W):
    R = _R
    Hp, Wp = H + 2 * R, W + 2 * R
    s_in = pred_ref[...]
    s_pad = jnp.concatenate(
        [jnp.pad(s_in[b], ((R, R), (R, R)), constant_values=-np.inf)
         for b in range(B)], axis=1)

    ypos = jax.lax.broadcasted_iota(jnp.int32, (Hp, Wp), 0)
    xpos = jax.lax.broadcasted_iota(jnp.int32, (Hp, Wp), 1)
    t1 = (-((ypos - R) * W + (xpos - R))).astype(jnp.float32)
    t_pad = jnp.concatenate([t1] * B, axis=1)

    thr = thr_ref[0]
    u0 = jnp.where(s_pad >= thr, 1.0, 0.0).astype(jnp.float32)
    k0 = jnp.zeros_like(u0)

    def wrow(ady):
        wv = w_ref[R + ady]
        return wv, jnp.clip(wv, 0, R)

    def lex_round(u, k):
        live = (u + k) > 0.5
        ms = jnp.where(live, s_pad, _NEG)
        mt = jnp.where(live, t_pad, _NEG)
        mk = k
        cur = (ms, mt, mk)
        st_a[0], st_b[0], st_c[0] = cur
        for r in range(1, R + 1):
            cur = _lexmax3(cur, (jnp.roll(ms, -r, 1), jnp.roll(mt, -r, 1),
                                 jnp.roll(mk, -r, 1)))
            cur = _lexmax3(cur, (jnp.roll(ms, r, 1), jnp.roll(mt, r, 1),
                                 jnp.roll(mk, r, 1)))
            st_a[r], st_b[r], st_c[r] = cur

        def row(ady):
            wv, wc = wrow(ady)
            ok = wv >= 0
            return (jnp.where(ok, st_a[wc], _NEG),
                    jnp.where(ok, st_b[wc], _NEG),
                    jnp.where(ok, st_c[wc], 0.0))
        acc = row(0)
        for ady in range(1, R + 1):
            sel_s, sel_t, sel_k = row(ady)
            acc = _lexmax3(acc, (jnp.roll(sel_s, -ady, 0),
                                 jnp.roll(sel_t, -ady, 0),
                                 jnp.roll(sel_k, -ady, 0)))
            acc = _lexmax3(acc, (jnp.roll(sel_s, ady, 0),
                                 jnp.roll(sel_t, ady, 0),
                                 jnp.roll(sel_k, ady, 0)))
        acc_s, acc_t, acc_k = acc
        gt = (acc_s > s_pad) | ((acc_s == s_pad) & (acc_t > t_pad))
        ub = u > 0.5
        new_k = ub & ~gt
        new_sup = ub & gt & (acc_k > 0.5)
        k2 = jnp.where(new_k, 1.0, k)
        u2 = jnp.where(new_k | new_sup, 0.0, u)
        return u2, k2

    def round_body(carry):
        u, k = carry
        live = (u + k) > 0.5
        ms = jnp.where(live, s_pad, _NEG)
        mks = jnp.where(k > 0.5, s_pad, _NEG)
        lf = jnp.roll(ms, 1, 1)
        rf = jnp.roll(ms, -1, 1)
        hkc = mks
        st_c[0] = hkc
        for r in range(1, R + 1):
            if r > 1:
                lf = jnp.maximum(lf, jnp.roll(ms, r, 1))
                rf = jnp.maximum(rf, jnp.roll(ms, -r, 1))
            h0 = jnp.maximum(lf, rf)
            st_a[r] = h0
            st_b[r] = jnp.maximum(h0, ms)
            hkc = jnp.maximum(hkc, jnp.maximum(jnp.roll(mks, -r, 1),
                                               jnp.roll(mks, r, 1)))
            st_c[r] = hkc
        st_b[0] = ms
        w0, w0c = wrow(0)
        acc_a = jnp.where(w0 >= 1, st_a[jnp.clip(w0, 1, R)], _NEG)
        acc_b = jnp.where(w0 >= 0, st_c[w0c], _NEG)
        up_a = dn_a = up_b = dn_b = None
        for ady in range(1, R + 1):
            wv, wc = wrow(ady)
            ok = wv >= 0
            sel_a = jnp.where(ok, st_b[wc], _NEG)
            sel_b = jnp.where(ok, st_c[wc], _NEG)
            ua, da = jnp.roll(sel_a, -ady, 0), jnp.roll(sel_a, ady, 0)
            ub_, db = jnp.roll(sel_b, -ady, 0), jnp.roll(sel_b, ady, 0)
            if up_a is None:
                up_a, dn_a, up_b, dn_b = ua, da, ub_, db
            else:
                up_a = jnp.maximum(up_a, ua)
                dn_a = jnp.maximum(dn_a, da)
                up_b = jnp.maximum(up_b, ub_)
                dn_b = jnp.maximum(dn_b, db)
        acc_a = jnp.maximum(acc_a, jnp.maximum(up_a, dn_a))
        acc_b = jnp.maximum(acc_b, jnp.maximum(up_b, dn_b))
        ub = u > 0.5
        tie_any = jnp.any(ub & ((acc_a == s_pad) | (acc_b == s_pad)))

        def fast(uk):
            uu, kk = uk
            new_k = ub & (acc_a < s_pad)
            new_sup = ub & (acc_b > s_pad)
            k2 = jnp.where(new_k, 1.0, kk)
            u2 = jnp.where(new_k | new_sup, 0.0, uu)
            return u2, k2

        def slow(uk):
            return lex_round(*uk)

        return jax.lax.cond(tie_any, slow, fast, (u, k))

    def cond(carry):
        return jnp.max(carry[0]) > 0.5

    _, k_fin = jax.lax.while_loop(cond, round_body, (u0, k0))
    keep = jnp.stack(
        [k_fin[R:R + H, b * Wp + R:b * Wp + R + W] for b in range(B)], axis=0)
    out_ref[...] = jnp.where(keep > 0.5, s_in, 0.0)


def kernel(pred_prob, size, threshold):
    B, H, W = pred_prob.shape
    size_f = jnp.asarray(size, jnp.float32)
    thr = jnp.asarray(threshold, jnp.float32).reshape(1)

    d = jnp.arange(-_R, _R + 1)
    ady = jnp.abs(d)[:, None].astype(jnp.float32)
    adx = jnp.abs(d)[None, :].astype(jnp.float32)
    inter = jnp.maximum(size_f - ady, 0.0) * jnp.maximum(size_f - adx, 0.0)
    iou_v = inter / (2.0 * size_f * size_f - inter)
    inc = iou_v > jnp.float32(0.1)
    w_row = jnp.max(jnp.where(inc, jnp.abs(d)[None, :], -1), axis=1).astype(jnp.int32)

    Hp, Wp = H + 2 * _R, W + 2 * _R
    body = functools.partial(_nms_body, B=B, H=H, W=W)
    return pl.pallas_call(
        body,
        out_shape=jax.ShapeDtypeStruct((B, H, W), jnp.float32),
        in_specs=[
            pl.BlockSpec(memory_space=pltpu.VMEM),
            pl.BlockSpec(memory_space=pltpu.SMEM),
            pl.BlockSpec(memory_space=pltpu.SMEM),
        ],
        out_specs=pl.BlockSpec(memory_space=pltpu.VMEM),
        scratch_shapes=[
            pltpu.VMEM((_R + 1, Hp, B * Wp), jnp.float32),
            pltpu.VMEM((_R + 1, Hp, B * Wp), jnp.float32),
            pltpu.VMEM((_R + 1, Hp, B * Wp), jnp.float32),
        ],
    )(pred_prob, w_row, thr)


# speedup vs baseline: 1.1064x; 1.1064x over previous
import functools

import jax
import jax.numpy as jnp
import numpy as np
from jax.experimental import pallas as pl
from jax.experimental.pallas import tpu as pltpu

_R = 6
_NEG = np.float32(-np.inf)


def _lexmax3(a, b):
    sa, ta, ka = a
    sb, tb, kb = b
    take = (sa > sb) | ((sa == sb) & (ta >= tb))
    return (jnp.where(take, sa, sb), jnp.where(take, ta, tb),
            jnp.where(take, ka, kb))


def _nms_body(pred_ref, w_ref, thr_ref, out_ref, st_a, st_b, st_c,
              *, B, H, W):
    R = _R
    SENT = R + 1
    Hp, Wp = H + 2 * R, W + 2 * R
    s_in = pred_ref[...]
    s_pad = jnp.concatenate(
        [jnp.pad(s_in[b], ((R, R), (R, R)), constant_values=-np.inf)
         for b in range(B)], axis=1)

    ypos = jax.lax.broadcasted_iota(jnp.int32, (Hp, Wp), 0)
    xpos = jax.lax.broadcasted_iota(jnp.int32, (Hp, Wp), 1)
    t1 = (-((ypos - R) * W + (xpos - R))).astype(jnp.float32)
    t_pad = jnp.concatenate([t1] * B, axis=1)

    neg_plane = jnp.full((Hp, B * Wp), _NEG, jnp.float32)
    st_a[SENT] = neg_plane
    st_b[SENT] = neg_plane
    st_c[SENT] = neg_plane

    thr = thr_ref[0]
    u0 = jnp.where(s_pad >= thr, 1.0, 0.0).astype(jnp.float32)
    k0 = jnp.zeros_like(u0)

    def widx(ady, lo):
        wv = w_ref[R + ady]
        return jnp.where(wv < lo, SENT, wv)

    def lex_round(uk):
        u, k = uk
        live = (u + k) > 0.5
        ms = jnp.where(live, s_pad, _NEG)
        mt = jnp.where(live, t_pad, _NEG)
        mk = k
        cur = (ms, mt, mk)
        st_a[0], st_b[0], st_c[0] = cur
        for r in range(1, R + 1):
            cur = _lexmax3(cur, (jnp.roll(ms, -r, 1), jnp.roll(mt, -r, 1),
                                 jnp.roll(mk, -r, 1)))
            cur = _lexmax3(cur, (jnp.roll(ms, r, 1), jnp.roll(mt, r, 1),
                                 jnp.roll(mk, r, 1)))
            st_a[r], st_b[r], st_c[r] = cur

        def row(ady):
            mi = widx(ady, 0)
            return st_a[mi], st_b[mi], st_c[mi]
        acc = row(0)
        for ady in range(1, R + 1):
            sel_s, sel_t, sel_k = row(ady)
            acc = _lexmax3(acc, (jnp.roll(sel_s, -ady, 0),
                                 jnp.roll(sel_t, -ady, 0),
                                 jnp.roll(sel_k, -ady, 0)))
            acc = _lexmax3(acc, (jnp.roll(sel_s, ady, 0),
                                 jnp.roll(sel_t, ady, 0),
                                 jnp.roll(sel_k, ady, 0)))
        acc_s, acc_t, acc_k = acc
        gt = (acc_s > s_pad) | ((acc_s == s_pad) & (acc_t > t_pad))
        ub = u > 0.5
        new_k = ub & ~gt
        new_sup = ub & gt & (acc_k > 0.5)
        k2 = jnp.where(new_k, 1.0, k)
        u2 = jnp.where(new_k | new_sup, 0.0, u)
        return u2, k2

    def fast_round(uk):
        u, k = uk
        live = (u + k) > 0.5
        ms = jnp.where(live, s_pad, _NEG)
        mks = jnp.where(k > 0.5, s_pad, _NEG)
        lf = jnp.roll(ms, 1, 1)
        rf = jnp.roll(ms, -1, 1)
        hkc = mks
        st_c[0] = hkc
        for r in range(1, R + 1):
            if r > 1:
                lf = jnp.maximum(lf, jnp.roll(ms, r, 1))
                rf = jnp.maximum(rf, jnp.roll(ms, -r, 1))
            h0 = jnp.maximum(lf, rf)
            st_a[r] = h0
            st_b[r] = jnp.maximum(h0, ms)
            hkc = jnp.maximum(hkc, jnp.maximum(jnp.roll(mks, -r, 1),
                                               jnp.roll(mks, r, 1)))
            st_c[r] = hkc
        st_b[0] = ms
        acc_a = st_a[widx(0, 1)]
        acc_b = st_c[widx(0, 0)]
        up_a = dn_a = up_b = dn_b = None
        for ady in range(1, R + 1):
            mi = widx(ady, 0)
            sel_a = st_b[mi]
            sel_b = st_c[mi]
            ua, da = jnp.roll(sel_a, -ady, 0), jnp.roll(sel_a, ady, 0)
            vb, db = jnp.roll(sel_b, -ady, 0), jnp.roll(sel_b, ady, 0)
            if up_a is None:
                up_a, dn_a, up_b, dn_b = ua, da, vb, db
            else:
                up_a = jnp.maximum(up_a, ua)
                dn_a = jnp.maximum(dn_a, da)
                up_b = jnp.maximum(up_b, vb)
                dn_b = jnp.maximum(dn_b, db)
        acc_a = jnp.maximum(acc_a, jnp.maximum(up_a, dn_a))
        acc_b = jnp.maximum(acc_b, jnp.maximum(up_b, dn_b))
        ub = u > 0.5
        new_k = ub & (acc_a < s_pad)
        new_sup = ub & (acc_b > s_pad)
        k2 = jnp.where(new_k, 1.0, k)
        u2 = jnp.where(new_k | new_sup, 0.0, u)
        return u2, k2

    def round_body(carry):
        u, k, n, lex_mode = carry
        u2, k2 = jax.lax.cond(lex_mode, lex_round, fast_round, (u, k))
        n2 = jnp.sum(u2)
        return u2, k2, n2, n2 == n

    def cond(carry):
        return carry[2] > 0.5

    n0 = jnp.sum(u0)
    _, k_fin, _, _ = jax.lax.while_loop(
        cond, round_body, (u0, k0, n0, jnp.zeros((), jnp.bool_)))
    keep = jnp.stack(
        [k_fin[R:R + H, b * Wp + R:b * Wp + R + W] for b in range(B)], axis=0)
    out_ref[...] = jnp.where(keep > 0.5, s_in, 0.0)


def kernel(pred_prob, size, threshold):
    B, H, W = pred_prob.shape
    size_f = jnp.asarray(size, jnp.float32)
    thr = jnp.asarray(threshold, jnp.float32).reshape(1)

    d = jnp.arange(-_R, _R + 1)
    ady = jnp.abs(d)[:, None].astype(jnp.float32)
    adx = jnp.abs(d)[None, :].astype(jnp.float32)
    inter = jnp.maximum(size_f - ady, 0.0) * jnp.maximum(size_f - adx, 0.0)
    iou_v = inter / (2.0 * size_f * size_f - inter)
    inc = iou_v > jnp.float32(0.1)
    w_row = jnp.max(jnp.where(inc, jnp.abs(d)[None, :], -1), axis=1).astype(jnp.int32)

    Hp, Wp = H + 2 * _R, W + 2 * _R
    body = functools.partial(_nms_body, B=B, H=H, W=W)
    return pl.pallas_call(
        body,
        out_shape=jax.ShapeDtypeStruct((B, H, W), jnp.float32),
        in_specs=[
            pl.BlockSpec(memory_space=pltpu.VMEM),
            pl.BlockSpec(memory_space=pltpu.SMEM),
            pl.BlockSpec(memory_space=pltpu.SMEM),
        ],
        out_specs=pl.BlockSpec(memory_space=pltpu.VMEM),
        scratch_shapes=[
            pltpu.VMEM((_R + 2, Hp, B * Wp), jnp.float32),
            pltpu.VMEM((_R + 2, Hp, B * Wp), jnp.float32),
            pltpu.VMEM((_R + 2, Hp, B * Wp), jnp.float32),
        ],
    )(pred_prob, w_row, thr)
